# baseline (speedup 1.0000x reference)
import functools

import jax
import jax.numpy as jnp
from jax import lax
from jax.experimental import pallas as pl
from jax.experimental.pallas import tpu as pltpu
from jax.experimental.pallas import tpu_sc as plsc

N = 10000
E = 320000
OUT = 128
H = 8
HD = 16

NC = 2
NS = 16
NW = NC * NS
EW = E // NW
C = 80
NCHUNK = EW // C
EWT = E // NS
NCHT = EWT // C
C1 = 64
EWT1 = 19968
NCHT1 = EWT1 // C1
NROWS = 624
NTAIL = N - NS * NROWS
SCHUNK = 48
NSC = NROWS // SCHUNK

BE = 1280
GE = E // BE

_MESH = plsc.VectorSubcoreMesh(core_axis_name="c", subcore_axis_name="s")


NBUF = 5


def _gather_body(xsrc, xdst, sidx, didx, src_out, dst_out, *scr):
    idxs = scr[0:NBUF]
    idxd = scr[NBUF:2 * NBUF]
    sr = scr[2 * NBUF:3 * NBUF]
    dr = scr[3 * NBUF:4 * NBUF]
    gsem = scr[4 * NBUF:5 * NBUF]
    wsem = scr[5 * NBUF:6 * NBUF]
    c = lax.axis_index("c")
    s = lax.axis_index("s")
    wid = s * NC + c
    base = wid * EW

    for b in range(NBUF):
        st = base + b * C
        pltpu.sync_copy(sidx.at[pl.ds(st, C)], idxs[b])
        pltpu.sync_copy(didx.at[pl.ds(st, C)], idxd[b])
        pltpu.async_copy(xsrc.at[idxs[b]], sr[b], gsem[b])
        pltpu.async_copy(xdst.at[idxd[b]], dr[b], gsem[b])

    def outer(g, carry):
        for b in range(NBUF):
            j = NBUF * g + b
            st = base + j * C
            pltpu.make_async_copy(xsrc.at[idxs[b]], sr[b], gsem[b]).wait()
            pltpu.make_async_copy(xdst.at[idxd[b]], dr[b], gsem[b]).wait()
            pltpu.async_copy(sr[b], src_out.at[pl.ds(st, C)], wsem[b])
            pltpu.async_copy(dr[b], dst_out.at[pl.ds(st, C)], wsem[b])

            @pl.when(g < NCHUNK // NBUF - 1)
            def _():
                nst = st + NBUF * C
                pltpu.sync_copy(sidx.at[pl.ds(nst, C)], idxs[b])
                pltpu.sync_copy(didx.at[pl.ds(nst, C)], idxd[b])
                pltpu.make_async_copy(sr[b], src_out.at[pl.ds(st, C)],
                                      wsem[b]).wait()
                pltpu.make_async_copy(dr[b], dst_out.at[pl.ds(st, C)],
                                      wsem[b]).wait()
                pltpu.async_copy(xsrc.at[idxs[b]], sr[b], gsem[b])
                pltpu.async_copy(xdst.at[idxd[b]], dr[b], gsem[b])

        return carry

    lax.fori_loop(0, NCHUNK // NBUF, outer, 0)
    for b in range(NBUF):
        st = base + (NCHUNK - NBUF + b) * C
        pltpu.make_async_copy(sr[b], src_out.at[pl.ds(st, C)], wsem[b]).wait()
        pltpu.make_async_copy(dr[b], dst_out.at[pl.ds(st, C)], wsem[b]).wait()


def _gather(x_src, x_dst, sidx, didx):
    k = pl.kernel(
        _gather_body,
        out_type=(
            jax.ShapeDtypeStruct((E, 128), jnp.float32),
            jax.ShapeDtypeStruct((E, 128), jnp.float32),
        ),
        mesh=_MESH,
        scratch_types=(
            [pltpu.VMEM((C,), jnp.int32)] * NBUF
            + [pltpu.VMEM((C,), jnp.int32)] * NBUF
            + [pltpu.VMEM((C, 128), jnp.float32)] * NBUF
            + [pltpu.VMEM((C, 128), jnp.float32)] * NBUF
            + [pltpu.SemaphoreType.DMA] * NBUF
            + [pltpu.SemaphoreType.DMA] * NBUF
        ),
    )
    return k(x_src, x_dst, sidx, didx)


def _fill_iota(ref, n, off):
    base = lax.iota(jnp.int32, 16)
    for t in range(n // 16):
        ref[pl.ds(16 * t, 16)] = base + (off + 16 * t)


def _zero_fill(ref, nrows, ncols):
    zv = jnp.zeros((16,), jnp.float32)

    def row(i, carry):
        for j in range(ncols // 16):
            ref[i, pl.ds(j * 16, 16)] = zv
        return carry

    lax.fori_loop(0, nrows, row, 0)


SNB = 2


def _scatter_body(ev, ep, didx, acc_out, s_out,
                  idxa, idxb, paya, payb, i1a, i1b, p1a, p1b, eva, evb,
                  stg, rix, tix,
                  rsem0, rsem1, asem0, asem1, accM):
    idx = (idxa, idxb)
    pay = (paya, payb)
    idx1 = (i1a, i1b)
    pay1 = (p1a, p1b)
    epv = (eva, evb)
    rsem = (rsem0, rsem1)
    asem = (asem0, asem1)
    c = lax.axis_index("c")
    s = lax.axis_index("s")

    _zero_fill(stg, SCHUNK, 128)

    def initk(k, carry):
        off = s * NROWS + k * SCHUNK
        _fill_iota(rix, SCHUNK, off)
        pltpu.sync_copy(stg, accM.at[rix])
        return carry

    lax.fori_loop(0, NSC, initk, 0)

    @pl.when(s == 0)
    def _():
        _fill_iota(tix, NTAIL, NS * NROWS)
        pltpu.sync_copy(stg.at[pl.ds(0, NTAIL)], accM.at[tix])

    plsc.subcore_barrier()

    base = s * EWT

    def make_loop(payload):
        def prime():
            for b in range(SNB):
                st = base + b * C
                pltpu.sync_copy(didx.at[pl.ds(st, C)], idx[b])
                pltpu.async_copy(payload.at[pl.ds(st, C)], pay[b], rsem[b])

        def outer(g, carry):
            for b in range(SNB):
                j = SNB * g + b
                st = base + j * C
                pltpu.make_async_copy(payload.at[pl.ds(st, C)], pay[b],
                                      rsem[b]).wait()
                pltpu.async_copy(pay[b], accM.at[idx[b]], asem[b], add=True)

                @pl.when(g < NCHT // SNB - 1)
                def _():
                    nst = st + SNB * C
                    pltpu.make_async_copy(pay[b], accM.at[idx[b]],
                                          asem[b]).wait()
                    pltpu.sync_copy(didx.at[pl.ds(nst, C)], idx[b])
                    pltpu.async_copy(payload.at[pl.ds(nst, C)], pay[b],
                                     rsem[b])

            return carry

        def drain():
            for b in range(SNB):
                pltpu.make_async_copy(pay[b], accM.at[idx[b]], asem[b]).wait()

        prime()
        lax.fori_loop(0, NCHT // SNB, outer, 0)
        drain()

    @pl.when(c == 0)
    def _():
        make_loop(ev)

    @pl.when(c == 1)
    def _():
        base1 = s * EWT1
        nch = NCHT1 + 8 * (s == NS - 1)
        for b in range(SNB):
            _zero_fill(pay1[b], C1, 128)

        def prime():
            for b in range(SNB):
                st = base1 + b * C1
                pltpu.sync_copy(didx.at[pl.ds(st, C1)], idx1[b])
                pltpu.async_copy(
                    ep.at[pl.ds(s * (EWT1 // 8) + b * (C1 // 8), C1 // 8)],
                    epv[b], rsem[b])

        def outer(g, carry):
            for b in range(SNB):
                j = SNB * g + b
                pltpu.make_async_copy(ep.at[pl.ds(0, C1 // 8)], epv[b],
                                      rsem[b]).wait()
                for i in range(C1):
                    pay1[b][i, pl.ds(0, 16)] = \
                        epv[b][i // 8, pl.ds(16 * (i % 8), 16)]
                pltpu.async_copy(pay1[b], accM.at[idx1[b]], asem[b], add=True)

                @pl.when(j + SNB < nch)
                def _():
                    pltpu.make_async_copy(pay1[b], accM.at[idx1[b]],
                                          asem[b]).wait()
                    nst = base1 + (j + SNB) * C1
                    pltpu.sync_copy(didx.at[pl.ds(nst, C1)], idx1[b])
                    pltpu.async_copy(
                        ep.at[pl.ds(s * (EWT1 // 8) + (j + SNB) * (C1 // 8),
                                    C1 // 8)],
                        epv[b], rsem[b])

            return carry

        prime()
        lax.fori_loop(0, nch // SNB, outer, 0)
        for b in range(SNB):
            pltpu.make_async_copy(pay1[b], accM.at[idx1[b]], asem[b]).wait()

    plsc.subcore_barrier()

    def outk(k, carry):
        off = s * NROWS + k * SCHUNK
        _fill_iota(rix, SCHUNK, off)
        pltpu.sync_copy(accM.at[rix], stg)

        @pl.when(c == 0)
        def _():
            pltpu.sync_copy(stg, acc_out.at[pl.ds(off, SCHUNK)])

        @pl.when(c == 1)
        def _():
            pltpu.sync_copy(stg, s_out.at[pl.ds(off, SCHUNK)])

        return carry

    lax.fori_loop(0, NSC, outk, 0)

    @pl.when(s == 0)
    def _():
        _fill_iota(tix, NTAIL, NS * NROWS)
        pltpu.sync_copy(accM.at[tix], stg.at[pl.ds(0, NTAIL)])

        @pl.when(c == 0)
        def _():
            pltpu.sync_copy(stg.at[pl.ds(0, NTAIL)],
                            acc_out.at[pl.ds(NS * NROWS, NTAIL)])

        @pl.when(c == 1)
        def _():
            pltpu.sync_copy(stg.at[pl.ds(0, NTAIL)],
                            s_out.at[pl.ds(NS * NROWS, NTAIL)])


def _scatter(ev, ep, didx):
    k = pl.kernel(
        _scatter_body,
        out_type=(
            jax.ShapeDtypeStruct((N, 128), jnp.float32),
            jax.ShapeDtypeStruct((N, 128), jnp.float32),
        ),
        mesh=_MESH,
        scratch_types=[
            pltpu.VMEM((C,), jnp.int32),
            pltpu.VMEM((C,), jnp.int32),
            pltpu.VMEM((C, 128), jnp.float32),
            pltpu.VMEM((C, 128), jnp.float32),
            pltpu.VMEM((C1,), jnp.int32),
            pltpu.VMEM((C1,), jnp.int32),
            pltpu.VMEM((C1, 128), jnp.float32),
            pltpu.VMEM((C1, 128), jnp.float32),
            pltpu.VMEM((C1 // 8, 128), jnp.float32),
            pltpu.VMEM((C1 // 8, 128), jnp.float32),
            pltpu.VMEM((SCHUNK, 128), jnp.float32),
            pltpu.VMEM((SCHUNK,), jnp.int32),
            pltpu.VMEM((NTAIL,), jnp.int32),
            pltpu.SemaphoreType.DMA,
            pltpu.SemaphoreType.DMA,
            pltpu.SemaphoreType.DMA,
            pltpu.SemaphoreType.DMA,
            pltpu.VMEM_SHARED((N, 128), jnp.float32),
        ],
    )
    return k(ev, ep, didx)


def _edge_kernel(xs_ref, xd_ref, ea_ref, W1s_ref, W1d_ref, W1e_ref, b1_ref,
                 Ws_ref, cs_ref, vW2_ref, vb2_ref, R_ref, ev_ref, e_ref):
    h1 = (jnp.dot(xs_ref[...], W1s_ref[...], preferred_element_type=jnp.float32)
          + jnp.dot(xd_ref[...], W1d_ref[...], preferred_element_type=jnp.float32)
          + jnp.dot(ea_ref[...], W1e_ref[...], preferred_element_type=jnp.float32)
          + b1_ref[...])
    h1 = jnp.maximum(h1, 0.0)
    h1k = h1[:, :128]
    h1v = h1[:, 128:]
    sc = jnp.dot(h1k, Ws_ref[...], preferred_element_type=jnp.float32) + cs_ref[...]
    e = jnp.exp(sc)
    v = (jnp.dot(h1v, vW2_ref[...], preferred_element_type=jnp.float32)
         + vb2_ref[...] + h1v)
    er = jnp.dot(e, R_ref[...], preferred_element_type=jnp.float32)
    ev_ref[...] = v * er
    e_ref[...] = e


def _edge_pass(src_g, dst_g, edge_attr, W1s, W1d, W1e, b1,
               Ws, cs, vW2, vb2, R):
    full = lambda shape: pl.BlockSpec(shape, lambda i: (0,) * len(shape))
    return pl.pallas_call(
        _edge_kernel,
        grid=(GE,),
        in_specs=[
            pl.BlockSpec((BE, 128), lambda i: (i, 0)),
            pl.BlockSpec((BE, 128), lambda i: (i, 0)),
            pl.BlockSpec((BE, 16), lambda i: (i, 0)),
            full((128, 256)),
            full((128, 256)),
            full((16, 256)),
            full((1, 256)),
            full((128, 8)),
            full((1, 8)),
            full((128, 128)),
            full((1, 128)),
            full((8, 128)),
        ],
        out_specs=[
            pl.BlockSpec((BE, 128), lambda i: (i, 0)),
            pl.BlockSpec((BE, 8), lambda i: (i, 0)),
        ],
        out_shape=[
            jax.ShapeDtypeStruct((E, 128), jnp.float32),
            jax.ShapeDtypeStruct((E, 8), jnp.float32),
        ],
    )(src_g, dst_g, edge_attr, W1s, W1d, W1e, b1, Ws, cs, vW2, vb2, R)


def _final_kernel(acc_ref, s_ref, R_ref, oW1_ref, ob1_ref, oW2_ref, ob2_ref,
                  out_ref):
    den = jnp.dot(s_ref[:, :8], R_ref[...],
                  preferred_element_type=jnp.float32) + 1e-16
    z = jnp.maximum(acc_ref[...] / den, 0.0)
    o1 = jnp.dot(z, oW1_ref[...], preferred_element_type=jnp.float32) \
        + ob1_ref[...] + z
    o1 = jnp.maximum(o1, 0.0)
    o2 = jnp.dot(o1, oW2_ref[...], preferred_element_type=jnp.float32) \
        + ob2_ref[...] + o1
    out_ref[...] = jnp.maximum(o2, 0.0)


def _final_pass(acc, sden, R, oW1, ob1, oW2, ob2):
    BN = 2000
    full = lambda shape: pl.BlockSpec(shape, lambda i: (0,) * len(shape))
    return pl.pallas_call(
        _final_kernel,
        grid=(N // BN,),
        in_specs=[
            pl.BlockSpec((BN, 128), lambda i: (i, 0)),
            pl.BlockSpec((BN, 128), lambda i: (i, 0)),
            full((8, 128)),
            full((128, 128)),
            full((1, 128)),
            full((128, 128)),
            full((1, 128)),
        ],
        out_specs=pl.BlockSpec((BN, 128), lambda i: (i, 0)),
        out_shape=jax.ShapeDtypeStruct((N, 128), jnp.float32),
    )(acc, sden, R, oW1, ob1, oW2, ob2)


def kernel(x_src, x_dst, edge_attr, edge_index, q, kW1, kb1, kW2, kb2,
           vW1, vb1, vW2, vb2, oW1, ob1, oW2, ob2):
    qv = q.reshape(OUT)
    qh = qv.reshape(H, HD)
    Ws = jnp.einsum("ihd,hd->ih", kW2.reshape(OUT, H, HD), qh)
    diag = jnp.zeros((OUT, H), jnp.float32).at[
        jnp.arange(OUT), jnp.arange(OUT) // HD].add(qv)
    Ws = (Ws + diag) * 0.25
    cs = ((kb2.reshape(H, HD) * qh).sum(-1) * 0.25).reshape(1, H)

    W1s = jnp.concatenate([kW1[:128], vW1[:128]], axis=1)
    W1d = jnp.concatenate([kW1[128:256], vW1[128:256]], axis=1)
    W1e = jnp.concatenate([kW1[256:], vW1[256:]], axis=1)
    b1 = jnp.concatenate([kb1, vb1]).reshape(1, 256)

    R = (jnp.arange(OUT)[None, :] // HD
         == jnp.arange(H)[:, None]).astype(jnp.float32)

    eidx = edge_index.astype(jnp.int32)
    sidx = eidx[0]
    didx = eidx[1]

    src_g, dst_g = _gather(x_src, x_dst, sidx, didx)
    ev, e = _edge_pass(src_g, dst_g, edge_attr, W1s, W1d, W1e, b1,
                       Ws, cs, vW2, vb2.reshape(1, 128), R)
    e3 = e.reshape(E // 8, 8, 8)
    ep = jnp.concatenate([e3, e3], axis=2).reshape(E // 8, 128)
    acc, sden = _scatter(ev, ep, didx)
    return _final_pass(acc, sden, R, oW1, ob1.reshape(1, 128),
                       oW2, ob2.reshape(1, 128))


# speedup vs baseline: 1.0495x; 1.0495x over previous
import functools

import jax
import jax.numpy as jnp
from jax import lax
from jax.experimental import pallas as pl
from jax.experimental.pallas import tpu as pltpu
from jax.experimental.pallas import tpu_sc as plsc

N = 10000
E = 320000
OUT = 128
H = 8
HD = 16

NC = 2
NS = 16
NW = NC * NS
EW = E // NW
C = 80
NCHUNK = EW // C
EWT = E // NS
NCHT = EWT // C
NROWS = 624
NTAIL = N - NS * NROWS
SCHUNK = 48
NSC = NROWS // SCHUNK

BE = 1280
GE = E // BE

_MESH = plsc.VectorSubcoreMesh(core_axis_name="c", subcore_axis_name="s")


NBUF = 5


def _gather_body(xsrc, xdst, sidx, didx, src_out, dst_out, *scr):
    idxs = scr[0:NBUF]
    idxd = scr[NBUF:2 * NBUF]
    sr = scr[2 * NBUF:3 * NBUF]
    dr = scr[3 * NBUF:4 * NBUF]
    gsem = scr[4 * NBUF:5 * NBUF]
    wsem = scr[5 * NBUF:6 * NBUF]
    c = lax.axis_index("c")
    s = lax.axis_index("s")
    wid = s * NC + c
    base = wid * EW

    for b in range(NBUF):
        st = base + b * C
        pltpu.sync_copy(sidx.at[pl.ds(st, C)], idxs[b])
        pltpu.sync_copy(didx.at[pl.ds(st, C)], idxd[b])
        pltpu.async_copy(xsrc.at[idxs[b]], sr[b], gsem[b])
        pltpu.async_copy(xdst.at[idxd[b]], dr[b], gsem[b])

    def outer(g, carry):
        for b in range(NBUF):
            j = NBUF * g + b
            st = base + j * C
            pltpu.make_async_copy(xsrc.at[idxs[b]], sr[b], gsem[b]).wait()
            pltpu.make_async_copy(xdst.at[idxd[b]], dr[b], gsem[b]).wait()
            pltpu.async_copy(sr[b], src_out.at[pl.ds(st, C)], wsem[b])
            pltpu.async_copy(dr[b], dst_out.at[pl.ds(st, C)], wsem[b])

            @pl.when(g < NCHUNK // NBUF - 1)
            def _():
                nst = st + NBUF * C
                pltpu.sync_copy(sidx.at[pl.ds(nst, C)], idxs[b])
                pltpu.sync_copy(didx.at[pl.ds(nst, C)], idxd[b])
                pltpu.make_async_copy(sr[b], src_out.at[pl.ds(st, C)],
                                      wsem[b]).wait()
                pltpu.make_async_copy(dr[b], dst_out.at[pl.ds(st, C)],
                                      wsem[b]).wait()
                pltpu.async_copy(xsrc.at[idxs[b]], sr[b], gsem[b])
                pltpu.async_copy(xdst.at[idxd[b]], dr[b], gsem[b])

        return carry

    lax.fori_loop(0, NCHUNK // NBUF, outer, 0)
    for b in range(NBUF):
        st = base + (NCHUNK - NBUF + b) * C
        pltpu.make_async_copy(sr[b], src_out.at[pl.ds(st, C)], wsem[b]).wait()
        pltpu.make_async_copy(dr[b], dst_out.at[pl.ds(st, C)], wsem[b]).wait()


def _gather(x_src, x_dst, sidx, didx):
    k = pl.kernel(
        _gather_body,
        out_type=(
            jax.ShapeDtypeStruct((E, 128), jnp.float32),
            jax.ShapeDtypeStruct((E, 128), jnp.float32),
        ),
        mesh=_MESH,
        scratch_types=(
            [pltpu.VMEM((C,), jnp.int32)] * NBUF
            + [pltpu.VMEM((C,), jnp.int32)] * NBUF
            + [pltpu.VMEM((C, 128), jnp.float32)] * NBUF
            + [pltpu.VMEM((C, 128), jnp.float32)] * NBUF
            + [pltpu.SemaphoreType.DMA] * NBUF
            + [pltpu.SemaphoreType.DMA] * NBUF
        ),
    )
    return k(x_src, x_dst, sidx, didx)


def _fill_iota(ref, n, off):
    base = lax.iota(jnp.int32, 16)
    for t in range(n // 16):
        ref[pl.ds(16 * t, 16)] = base + (off + 16 * t)


def _zero_fill(ref, nrows, ncols):
    zv = jnp.zeros((16,), jnp.float32)

    def row(i, carry):
        for j in range(ncols // 16):
            ref[i, pl.ds(j * 16, 16)] = zv
        return carry

    lax.fori_loop(0, nrows, row, 0)


SNB = 2


def _scatter_body(ev, er, didx, acc_out, s_out,
                  idxa, idxb, paya, payb, stg, rix, tix,
                  rsem0, rsem1, asem0, asem1, accM):
    idx = (idxa, idxb)
    pay = (paya, payb)
    rsem = (rsem0, rsem1)
    asem = (asem0, asem1)
    c = lax.axis_index("c")
    s = lax.axis_index("s")

    _zero_fill(stg, SCHUNK, 128)

    def initk(k, carry):
        off = s * NROWS + k * SCHUNK
        _fill_iota(rix, SCHUNK, off)
        pltpu.sync_copy(stg, accM.at[rix])
        return carry

    lax.fori_loop(0, NSC, initk, 0)

    @pl.when(s == 0)
    def _():
        _fill_iota(tix, NTAIL, NS * NROWS)
        pltpu.sync_copy(stg.at[pl.ds(0, NTAIL)], accM.at[tix])

    plsc.subcore_barrier()

    base = s * EWT

    def make_loop(payload):
        def prime():
            for b in range(SNB):
                st = base + b * C
                pltpu.sync_copy(didx.at[pl.ds(st, C)], idx[b])
                pltpu.async_copy(payload.at[pl.ds(st, C)], pay[b], rsem[b])

        def outer(g, carry):
            for b in range(SNB):
                j = SNB * g + b
                st = base + j * C
                pltpu.make_async_copy(payload.at[pl.ds(st, C)], pay[b],
                                      rsem[b]).wait()
                pltpu.async_copy(pay[b], accM.at[idx[b]], asem[b], add=True)

                @pl.when(g < NCHT // SNB - 1)
                def _():
                    nst = st + SNB * C
                    pltpu.make_async_copy(pay[b], accM.at[idx[b]],
                                          asem[b]).wait()
                    pltpu.sync_copy(didx.at[pl.ds(nst, C)], idx[b])
                    pltpu.async_copy(payload.at[pl.ds(nst, C)], pay[b],
                                     rsem[b])

            return carry

        def drain():
            for b in range(SNB):
                pltpu.make_async_copy(pay[b], accM.at[idx[b]], asem[b]).wait()

        prime()
        lax.fori_loop(0, NCHT // SNB, outer, 0)
        drain()

    @pl.when(c == 0)
    def _():
        make_loop(ev)

    @pl.when(c == 1)
    def _():
        make_loop(er)

    plsc.subcore_barrier()

    def outk(k, carry):
        off = s * NROWS + k * SCHUNK
        _fill_iota(rix, SCHUNK, off)
        pltpu.sync_copy(accM.at[rix], stg)

        @pl.when(c == 0)
        def _():
            pltpu.sync_copy(stg, acc_out.at[pl.ds(off, SCHUNK)])

        @pl.when(c == 1)
        def _():
            pltpu.sync_copy(stg, s_out.at[pl.ds(off, SCHUNK)])

        return carry

    lax.fori_loop(0, NSC, outk, 0)

    @pl.when(s == 0)
    def _():
        _fill_iota(tix, NTAIL, NS * NROWS)
        pltpu.sync_copy(accM.at[tix], stg.at[pl.ds(0, NTAIL)])

        @pl.when(c == 0)
        def _():
            pltpu.sync_copy(stg.at[pl.ds(0, NTAIL)],
                            acc_out.at[pl.ds(NS * NROWS, NTAIL)])

        @pl.when(c == 1)
        def _():
            pltpu.sync_copy(stg.at[pl.ds(0, NTAIL)],
                            s_out.at[pl.ds(NS * NROWS, NTAIL)])


def _scatter(ev, er, didx):
    k = pl.kernel(
        _scatter_body,
        out_type=(
            jax.ShapeDtypeStruct((N, 128), jnp.float32),
            jax.ShapeDtypeStruct((N, 128), jnp.float32),
        ),
        mesh=_MESH,
        scratch_types=[
            pltpu.VMEM((C,), jnp.int32),
            pltpu.VMEM((C,), jnp.int32),
            pltpu.VMEM((C, 128), jnp.float32),
            pltpu.VMEM((C, 128), jnp.float32),
            pltpu.VMEM((SCHUNK, 128), jnp.float32),
            pltpu.VMEM((SCHUNK,), jnp.int32),
            pltpu.VMEM((NTAIL,), jnp.int32),
            pltpu.SemaphoreType.DMA,
            pltpu.SemaphoreType.DMA,
            pltpu.SemaphoreType.DMA,
            pltpu.SemaphoreType.DMA,
            pltpu.VMEM_SHARED((N, 128), jnp.float32),
        ],
    )
    return k(ev, er, didx)


def _edge_kernel(xs_ref, xd_ref, ea_ref, W1s_ref, W1d_ref, W1e_ref, b1_ref,
                 Ws_ref, cs_ref, vW2_ref, vb2_ref, R_ref, ev_ref, e_ref):
    bf = jnp.bfloat16
    h1 = (jnp.dot(xs_ref[...].astype(bf), W1s_ref[...],
                  preferred_element_type=jnp.float32)
          + jnp.dot(xd_ref[...].astype(bf), W1d_ref[...],
                    preferred_element_type=jnp.float32)
          + jnp.dot(ea_ref[...], W1e_ref[...],
                    preferred_element_type=jnp.float32)
          + b1_ref[...])
    h1 = jnp.maximum(h1, 0.0)
    h1k = h1[:, :128]
    h1v = h1[:, 128:]
    sc = jnp.dot(h1k, Ws_ref[...], preferred_element_type=jnp.float32) + cs_ref[...]
    e = jnp.exp(sc)
    v = (jnp.dot(h1v.astype(bf), vW2_ref[...],
                 preferred_element_type=jnp.float32)
         + vb2_ref[...] + h1v)
    er = jnp.dot(e, R_ref[...], preferred_element_type=jnp.float32)
    ev_ref[...] = v * er
    e_ref[...] = er


def _edge_pass(src_g, dst_g, edge_attr, W1s, W1d, W1e, b1,
               Ws, cs, vW2, vb2, R):
    full = lambda shape: pl.BlockSpec(shape, lambda i: (0,) * len(shape))
    return pl.pallas_call(
        _edge_kernel,
        grid=(GE,),
        in_specs=[
            pl.BlockSpec((BE, 128), lambda i: (i, 0)),
            pl.BlockSpec((BE, 128), lambda i: (i, 0)),
            pl.BlockSpec((BE, 16), lambda i: (i, 0)),
            full((128, 256)),
            full((128, 256)),
            full((16, 256)),
            full((1, 256)),
            full((128, 8)),
            full((1, 8)),
            full((128, 128)),
            full((1, 128)),
            full((8, 128)),
        ],
        out_specs=[
            pl.BlockSpec((BE, 128), lambda i: (i, 0)),
            pl.BlockSpec((BE, 128), lambda i: (i, 0)),
        ],
        out_shape=[
            jax.ShapeDtypeStruct((E, 128), jnp.float32),
            jax.ShapeDtypeStruct((E, 128), jnp.float32),
        ],
    )(src_g, dst_g, edge_attr, W1s, W1d, W1e, b1, Ws, cs, vW2, vb2, R)


def _final_kernel(acc_ref, s_ref, oW1_ref, ob1_ref, oW2_ref, ob2_ref,
                  out_ref):
    den = s_ref[...] + 1e-16
    z = jnp.maximum(acc_ref[...] / den, 0.0)
    o1 = jnp.dot(z, oW1_ref[...], preferred_element_type=jnp.float32) \
        + ob1_ref[...] + z
    o1 = jnp.maximum(o1, 0.0)
    o2 = jnp.dot(o1, oW2_ref[...], preferred_element_type=jnp.float32) \
        + ob2_ref[...] + o1
    out_ref[...] = jnp.maximum(o2, 0.0)


def _final_pass(acc, sden, oW1, ob1, oW2, ob2):
    BN = 2000
    full = lambda shape: pl.BlockSpec(shape, lambda i: (0,) * len(shape))
    return pl.pallas_call(
        _final_kernel,
        grid=(N // BN,),
        in_specs=[
            pl.BlockSpec((BN, 128), lambda i: (i, 0)),
            pl.BlockSpec((BN, 128), lambda i: (i, 0)),
            full((128, 128)),
            full((1, 128)),
            full((128, 128)),
            full((1, 128)),
        ],
        out_specs=pl.BlockSpec((BN, 128), lambda i: (i, 0)),
        out_shape=jax.ShapeDtypeStruct((N, 128), jnp.float32),
    )(acc, sden, oW1, ob1, oW2, ob2)


def kernel(x_src, x_dst, edge_attr, edge_index, q, kW1, kb1, kW2, kb2,
           vW1, vb1, vW2, vb2, oW1, ob1, oW2, ob2):
    qv = q.reshape(OUT)
    qh = qv.reshape(H, HD)
    Ws = jnp.einsum("ihd,hd->ih", kW2.reshape(OUT, H, HD), qh)
    diag = jnp.zeros((OUT, H), jnp.float32).at[
        jnp.arange(OUT), jnp.arange(OUT) // HD].add(qv)
    Ws = (Ws + diag) * 0.25
    cs = ((kb2.reshape(H, HD) * qh).sum(-1) * 0.25).reshape(1, H)

    W1s = jnp.concatenate([kW1[:128], vW1[:128]], axis=1)
    W1d = jnp.concatenate([kW1[128:256], vW1[128:256]], axis=1)
    W1e = jnp.concatenate([kW1[256:], vW1[256:]], axis=1)
    b1 = jnp.concatenate([kb1, vb1]).reshape(1, 256)

    R = (jnp.arange(OUT)[None, :] // HD
         == jnp.arange(H)[:, None]).astype(jnp.float32)

    eidx = edge_index.astype(jnp.int32)
    sidx = eidx[0]
    didx = eidx[1]

    src_g, dst_g = _gather(x_src, x_dst, sidx, didx)
    ev, er = _edge_pass(src_g, dst_g, edge_attr,
                        W1s.astype(jnp.bfloat16), W1d.astype(jnp.bfloat16),
                        W1e, b1, Ws, cs, vW2.astype(jnp.bfloat16),
                        vb2.reshape(1, 128), R)
    acc, sden = _scatter(ev, er, didx)
    return _final_pass(acc, sden, oW1, ob1.reshape(1, 128),
                       oW2, ob2.reshape(1, 128))
